# baseline (speedup 1.0000x reference)
import functools

import jax
import jax.numpy as jnp
from jax import lax
from jax.experimental import pallas as pl
from jax.experimental.pallas import tpu as pltpu
from jax.experimental.pallas import tpu_sc as plsc

L = 16


@functools.lru_cache(maxsize=None)
def _make_sc_lookup(batch, seqlen, embed, vocab):
    info = plsc.get_sparse_core_info()
    nw = info.num_cores * info.num_subcores
    assert batch % nw == 0
    rows_per_w = batch // nw
    mesh = plsc.VectorSubcoreMesh(core_axis_name="c", subcore_axis_name="s")

    @functools.partial(
        pl.kernel,
        mesh=mesh,
        compiler_params=pltpu.CompilerParams(use_tc_tiling_on_sc=False),
        out_type=jax.ShapeDtypeStruct((batch * seqlen, embed), jnp.float32),
        scratch_types=[
            pltpu.VMEM((seqlen,), jnp.int32),
            pltpu.VMEM((seqlen, embed), jnp.float32),
            pltpu.VMEM((seqlen, embed), jnp.float32),
            pltpu.SemaphoreType.DMA,
        ],
    )
    def k(x_hbm, tok_hbm, pos_hbm, out_hbm, idx_v, rows_v, pos_v, sem):
        wid = lax.axis_index("s") * info.num_cores + lax.axis_index("c")
        pltpu.sync_copy(pos_hbm, pos_v)

        def body(i, _):
            base = (wid * rows_per_w + i) * seqlen
            pltpu.sync_copy(x_hbm.at[pl.ds(base, seqlen)], idx_v)
            pltpu.async_copy(tok_hbm.at[idx_v], rows_v, sem).wait()

            def add_row(r, _):
                for c in range(embed // L):
                    sl = pl.ds(c * L, L)
                    rows_v[r, sl] = rows_v[r, sl] + pos_v[r, sl]
                return 0

            lax.fori_loop(0, seqlen, add_row, 0)
            pltpu.sync_copy(rows_v, out_hbm.at[pl.ds(base, seqlen)])
            return 0

        lax.fori_loop(0, rows_per_w, body, 0)

    return k


def kernel(x, token_table, pos_table):
    batch, seqlen = x.shape
    vocab, embed = token_table.shape
    k = _make_sc_lookup(batch, seqlen, embed, vocab)
    out = k(x.reshape(-1).astype(jnp.int32), token_table, pos_table)
    return out.reshape(batch, seqlen, embed)


# speedup vs baseline: 1.2100x; 1.2100x over previous
import functools

import jax
import jax.numpy as jnp
from jax import lax
from jax.experimental import pallas as pl
from jax.experimental.pallas import tpu as pltpu
from jax.experimental.pallas import tpu_sc as plsc

NB = 4


@functools.lru_cache(maxsize=None)
def _make_sc_lookup(batch, seqlen, embed, vocab):
    info = plsc.get_sparse_core_info()
    nw = info.num_cores * info.num_subcores
    assert batch % nw == 0
    chunks = batch // nw
    assert chunks % NB == 0 and chunks >= NB
    mesh = plsc.VectorSubcoreMesh(core_axis_name="c", subcore_axis_name="s")

    @functools.partial(
        pl.kernel,
        mesh=mesh,
        compiler_params=pltpu.CompilerParams(use_tc_tiling_on_sc=False),
        out_type=jax.ShapeDtypeStruct((batch * seqlen, embed), jnp.float32),
        scratch_types=(
            [pltpu.VMEM((seqlen, embed), jnp.float32) for _ in range(NB)]
            + [pltpu.VMEM((seqlen,), jnp.int32) for _ in range(NB)]
            + [pltpu.VMEM_SHARED((seqlen, embed), jnp.float32)]
            + [pltpu.SemaphoreType.DMA for _ in range(4 * NB)]
        ),
    )
    def k(x_hbm, tok_hbm, pos_hbm, out_hbm, *scratch):
        rows = scratch[:NB]
        idxs = scratch[NB:2 * NB]
        pos_sh = scratch[2 * NB]
        isem = scratch[2 * NB + 1:2 * NB + 1 + NB]
        nsem = scratch[2 * NB + 1 + NB:2 * NB + 1 + 2 * NB]
        gsem = scratch[2 * NB + 1 + 2 * NB:2 * NB + 1 + 3 * NB]
        ssem = scratch[2 * NB + 1 + 3 * NB:]

        wid = lax.axis_index("s") * info.num_cores + lax.axis_index("c")
        row0 = wid * chunks

        @pl.when(lax.axis_index("s") == 0)
        def _():
            pltpu.sync_copy(pos_hbm, pos_sh)

        plsc.subcore_barrier()

        def x_slice(i):
            return x_hbm.at[pl.ds((row0 + i) * seqlen, seqlen)]

        def out_slice(i):
            return out_hbm.at[pl.ds((row0 + i) * seqlen, seqlen)]

        def fetch(i, b):
            @pl.when(i < chunks)
            def _():
                @pl.when(i >= NB)
                def _():
                    pltpu.make_async_copy(rows[b], out_slice(i - NB), ssem[b]).wait()

                pltpu.async_copy(x_slice(i), idxs[b], isem[b])
                pltpu.async_copy(pos_sh, rows[b], nsem[b])

        def gather(i, b):
            @pl.when(jnp.logical_and(i >= 0, i < chunks))
            def _():
                pltpu.make_async_copy(x_slice(i), idxs[b], isem[b]).wait()
                pltpu.make_async_copy(pos_sh, rows[b], nsem[b]).wait()
                pltpu.async_copy(tok_hbm.at[idxs[b]], rows[b], gsem[b], add=True)

        def store(i, b):
            @pl.when(jnp.logical_and(i >= 0, i < chunks))
            def _():
                pltpu.make_async_copy(tok_hbm.at[idxs[b]], rows[b], gsem[b]).wait()
                pltpu.async_copy(rows[b], out_slice(i), ssem[b])

        def visit_group(kk, _):
            for j in range(NB):
                v = NB * kk + j - 2
                fetch(v + 2, j)
                gather(v + 1, (j + 3) % NB)
                store(v, (j + 2) % NB)
            return 0

        lax.fori_loop(0, chunks // NB + 1, visit_group, 0)

        for b in range(NB):
            pltpu.make_async_copy(rows[b], out_slice(0), ssem[b]).wait()

    return k


def kernel(x, token_table, pos_table):
    batch, seqlen = x.shape
    vocab, embed = token_table.shape
    k = _make_sc_lookup(batch, seqlen, embed, vocab)
    out = k(x.reshape(-1).astype(jnp.int32), token_table, pos_table)
    return out.reshape(batch, seqlen, embed)
